# Initial kernel scaffold; baseline (speedup 1.0000x reference)
import jax
import jax.numpy as jnp
from jax.experimental import pallas as pl


def kernel(Xs, h0, ru_W1, ru_b1, ru_W2, ru_b2, ru_W3, ru_b3, rh_W1, rh_b1, rh_W2, rh_b2, rh_W3, rh_b3, edge_index):
    raise NotImplementedError("write your pallas kernel here")


# speedup vs baseline: 2.7576x; 2.7576x over previous
import jax
import jax.numpy as jnp
from jax import lax
from jax.experimental import pallas as pl
from jax.experimental.pallas import tpu as pltpu
from jax.experimental.pallas import tpu_sc as plsc

N = 10000
DH = 128
DQ = 64
T = 4
TILES = 16
CORES = 2
GE = 128
G = 160
EPAD = TILES * G * GE
ZROWS = 632
NPAD = TILES * ZROWS
RBLK = 1000

_MESH = plsc.VectorSubcoreMesh(core_axis_name="c", subcore_axis_name="s")


def _agg_body(tt, srcg, dstg, zrows, oo, src_v, dst_v, rows_v, acc, sem0, sem1):
    c = lax.axis_index("c")
    s = lax.axis_index("s")
    pltpu.sync_copy(dstg.at[s], dst_v)
    for p in range(2):
        q = 2 * c + p
        pltpu.sync_copy(srcg.at[q, s], src_v)
        pltpu.sync_copy(zrows, acc.at[pl.ds(s * ZROWS, ZROWS)])
        plsc.subcore_barrier()

        def step(i, carry):
            g0 = 2 * i
            g1 = g0 + 1
            d0 = pltpu.async_copy(tt.at[src_v.at[g0]], rows_v.at[0], sem0)
            d1 = pltpu.async_copy(tt.at[src_v.at[g1]], rows_v.at[1], sem1)
            d0.wait()
            pltpu.sync_copy(rows_v.at[0], acc.at[dst_v.at[g0]], add=True)
            d1.wait()
            pltpu.sync_copy(rows_v.at[1], acc.at[dst_v.at[g1]], add=True)
            return carry

        lax.fori_loop(0, G // 2, step, 0)
        plsc.subcore_barrier()
        pltpu.sync_copy(acc.at[pl.ds(s * ZROWS, ZROWS)],
                        oo.at[q, pl.ds(s * ZROWS, ZROWS)])


_agg = pl.kernel(
    _agg_body,
    out_type=jax.ShapeDtypeStruct((4, NPAD, DQ), jnp.float32),
    mesh=_MESH,
    compiler_params=pltpu.CompilerParams(use_tc_tiling_on_sc=False),
    scratch_types=[
        pltpu.VMEM((G, GE), jnp.int32),
        pltpu.VMEM((G, GE), jnp.int32),
        pltpu.VMEM((2, GE, DQ), jnp.float32),
        pltpu.VMEM_SHARED((NPAD, DQ), jnp.float32),
        pltpu.SemaphoreType.DMA,
        pltpu.SemaphoreType.DMA,
    ],
)


def _deg_body(degidx, onerow, z16, dd, idx_v, ones_v, acc):
    c = lax.axis_index("c")
    s = lax.axis_index("s")
    pltpu.sync_copy(degidx.at[c, s], idx_v)
    pltpu.sync_copy(onerow, ones_v)
    pltpu.sync_copy(z16, acc.at[pl.ds(s * ZROWS, ZROWS)])
    plsc.subcore_barrier()

    def step(g, carry):
        pltpu.sync_copy(ones_v, acc.at[idx_v.at[g]], add=True)
        return carry

    lax.fori_loop(0, G, step, 0)
    plsc.subcore_barrier()
    pltpu.sync_copy(acc.at[pl.ds(s * ZROWS, ZROWS)],
                    dd.at[c, pl.ds(s * ZROWS, ZROWS)])


_deg = pl.kernel(
    _deg_body,
    out_type=jax.ShapeDtypeStruct((CORES, NPAD, 16), jnp.float32),
    mesh=_MESH,
    compiler_params=pltpu.CompilerParams(use_tc_tiling_on_sc=False),
    scratch_types=[
        pltpu.VMEM((G, GE), jnp.int32),
        pltpu.VMEM((GE, 16), jnp.float32),
        pltpu.VMEM_SHARED((NPAD, 16), jnp.float32),
    ],
)


def _nrm(d):
    return lax.rsqrt(jnp.maximum(d, 1.0))


def _matpair(aq, di, W, b):
    nd = _nrm(di[...])
    Wv = W[...]
    t = b[...]
    for q in range(4):
        t = t + jnp.dot(aq[q] * nd, Wv[DQ * q:DQ * (q + 1)],
                        preferred_element_type=jnp.float32)
    return t


def _quarters(o, gs):
    for q in range(4):
        o[q] = gs[:, DQ * q:DQ * (q + 1)]


def _mid_body(a0, a1, a2, a3, di, do, W, b, o):
    g = jnp.maximum(_matpair((a0[0], a1[0], a2[0], a3[0]), di, W, b), 0.0)
    _quarters(o, g * _nrm(do[...]))


def _ru3_body(a0, a1, a2, a3, di, do, W, b, h, z_o, hrs_o):
    ru = jax.nn.sigmoid(_matpair((a0[0], a1[0], a2[0], a3[0]), di, W, b))
    r = ru[:, :DH]
    z_o[...] = ru[:, DH:]
    hrs = h[...] * r * _nrm(do[...])
    hrs_o[0] = hrs[:, :DQ]
    hrs_o[1] = hrs[:, DQ:]


def _rh3_body(a0, a1, a2, a3, di, do, W, b, h, z, hn_o, hns_o):
    hc = jnp.tanh(_matpair((a0[0], a1[0], a2[0], a3[0]), di, W, b))
    zv = z[...]
    hn = zv * h[...] + (1.0 - zv) * hc
    hn_o[...] = hn
    hns = hn * _nrm(do[...])
    hns_o[0] = hns[:, :DQ]
    hns_o[1] = hns[:, DQ:]


def _pre_body(h0, Xs, do, h0s_o, Xss_o):
    ns = _nrm(do[...])
    h0s = h0[...] * ns
    h0s_o[0] = h0s[:, :DQ]
    h0s_o[1] = h0s[:, DQ:]
    xs = Xs[...] * ns[None]
    Xss_o[:, 0] = xs[..., :DQ]
    Xss_o[:, 1] = xs[..., DQ:]


_row = pl.BlockSpec((RBLK, DH), lambda i: (i, 0))
_col1 = pl.BlockSpec((RBLK, 1), lambda i: (i, 0))
_q = [pl.BlockSpec((1, RBLK, DQ), lambda i, q=q: (q, i, 0)) for q in range(4)]
_q4 = pl.BlockSpec((4, RBLK, DQ), lambda i: (0, i, 0))
_h2 = pl.BlockSpec((2, RBLK, DQ), lambda i: (0, i, 0))
_full = lambda shape: pl.BlockSpec(shape, lambda i: tuple(0 for _ in shape))
_GRID = N // RBLK

_mid = pl.pallas_call(
    _mid_body,
    grid=(_GRID,),
    in_specs=[*_q, _col1, _col1, _full((2 * DH, 2 * DH)), _full((1, 2 * DH))],
    out_specs=[_q4],
    out_shape=[jax.ShapeDtypeStruct((4, N, DQ), jnp.float32)],
)

_ru3 = pl.pallas_call(
    _ru3_body,
    grid=(_GRID,),
    in_specs=[*_q, _col1, _col1, _full((2 * DH, 2 * DH)),
              _full((1, 2 * DH)), _row],
    out_specs=[_row, _h2],
    out_shape=[jax.ShapeDtypeStruct((N, DH), jnp.float32),
               jax.ShapeDtypeStruct((2, N, DQ), jnp.float32)],
)

_rh3 = pl.pallas_call(
    _rh3_body,
    grid=(_GRID,),
    in_specs=[*_q, _col1, _col1, _full((2 * DH, DH)),
              _full((1, DH)), _row, _row],
    out_specs=[_row, _h2],
    out_shape=[jax.ShapeDtypeStruct((N, DH), jnp.float32),
               jax.ShapeDtypeStruct((2, N, DQ), jnp.float32)],
)

_pre = pl.pallas_call(
    _pre_body,
    grid=(_GRID,),
    in_specs=[_row, pl.BlockSpec((T, RBLK, DH), lambda i: (0, i, 0)), _col1],
    out_specs=[_h2, pl.BlockSpec((T, 2, RBLK, DQ), lambda i: (0, 0, i, 0))],
    out_shape=[jax.ShapeDtypeStruct((2, N, DQ), jnp.float32),
               jax.ShapeDtypeStruct((T, 2, N, DQ), jnp.float32)],
)


def kernel(Xs, h0, ru_W1, ru_b1, ru_W2, ru_b2, ru_W3, ru_b3,
           rh_W1, rh_b1, rh_W2, rh_b2, rh_W3, rh_b3, edge_index):
    src = edge_index[0].astype(jnp.int32)
    dst = edge_index[1].astype(jnp.int32)
    E = src.shape[0]
    pad = EPAD - E
    src_p0 = jnp.concatenate([src, jnp.zeros((pad,), jnp.int32)])
    src_pN = jnp.concatenate([src, jnp.full((pad,), N, jnp.int32)])
    dst_p = jnp.concatenate([dst, jnp.full((pad,), N, jnp.int32)])
    srcg = jnp.stack([src_p0 + q * N for q in range(4)])
    srcg = srcg.reshape(4, TILES, G, GE)
    dstg = dst_p.reshape(TILES, G, GE)
    degidx = jnp.stack([src_pN, dst_p]).reshape(CORES, TILES, G, GE)

    zrows = jnp.zeros((ZROWS, DQ), jnp.float32)
    z16 = jnp.zeros((ZROWS, 16), jnp.float32)
    onerow = jnp.zeros((GE, 16), jnp.float32).at[:, 0].set(1.0)

    dd = _deg(degidx, onerow, z16)
    deg_out = dd[0, :N, 0:1]
    deg_in = dd[1, :N, 0:1]

    b_ru1 = ru_b1.reshape(1, -1)
    b_ru2 = ru_b2.reshape(1, -1)
    b_ru3 = ru_b3.reshape(1, -1)
    b_rh1 = rh_b1.reshape(1, -1)
    b_rh2 = rh_b2.reshape(1, -1)
    b_rh3 = rh_b3.reshape(1, -1)

    h_s, Xss = _pre(h0, Xs, deg_out)

    def agg(table):
        a = _agg(table.reshape(-1, DQ), srcg, dstg, zrows)
        return (a, a, a, a)

    def gnn2(t0, t1, W1, b1, W2, b2):
        a = agg(jnp.concatenate([t0, t1], axis=0))
        (gg,) = _mid(*a, deg_in, deg_out, W1, b1)
        a = agg(gg)
        (gg,) = _mid(*a, deg_in, deg_out, W2, b2)
        return agg(gg)

    h = h0
    hs = []
    for t in range(T):
        xst = Xss[t]
        a = gnn2(h_s, xst, ru_W1, b_ru1, ru_W2, b_ru2)
        z, hrs = _ru3(*a, deg_in, deg_out, ru_W3, b_ru3, h)
        a = gnn2(hrs, xst, rh_W1, b_rh1, rh_W2, b_rh2)
        h, h_s = _rh3(*a, deg_in, deg_out, rh_W3, b_rh3, h, z)
        hs.append(h)
    return jnp.stack(hs)


# speedup vs baseline: 3.0885x; 1.1200x over previous
import jax
import jax.numpy as jnp
from jax import lax
from jax.experimental import pallas as pl
from jax.experimental.pallas import tpu as pltpu
from jax.experimental.pallas import tpu_sc as plsc

N = 10000
DH = 128
DQ = 64
T = 4
TILES = 16
CORES = 2
GE = 128
G = 160
EPAD = TILES * G * GE
ZROWS = 632
NPAD = TILES * ZROWS
RBLK = 1000

_MESH = plsc.VectorSubcoreMesh(core_axis_name="c", subcore_axis_name="s")


NBUF = 5


def _agg_body(tt, srcg, dstg, zrows, oo, src_v, dst_v, rows_v, acc, gsem, ssem):
    c = lax.axis_index("c")
    s = lax.axis_index("s")
    pltpu.sync_copy(dstg.at[s], dst_v)
    for p in range(2):
        q = 2 * c + p
        pltpu.sync_copy(srcg.at[q, s], src_v)
        pltpu.sync_copy(zrows, acc.at[pl.ds(s * ZROWS, ZROWS)])
        plsc.subcore_barrier()

        def macro(m, carry):
            base = NBUF * m
            gds = [
                pltpu.async_copy(tt.at[src_v.at[base + k]], rows_v.at[k],
                                 gsem.at[k])
                for k in range(NBUF)
            ]
            sds = []
            for k in range(NBUF):
                gds[k].wait()
                sds.append(pltpu.async_copy(
                    rows_v.at[k], acc.at[dst_v.at[base + k]], ssem, add=True))
            for d in sds:
                d.wait()
            return carry

        lax.fori_loop(0, G // NBUF, macro, 0)
        plsc.subcore_barrier()
        pltpu.sync_copy(acc.at[pl.ds(s * ZROWS, ZROWS)],
                        oo.at[q, pl.ds(s * ZROWS, ZROWS)])


_agg = pl.kernel(
    _agg_body,
    out_type=jax.ShapeDtypeStruct((4, NPAD, DQ), jnp.float32),
    mesh=_MESH,
    compiler_params=pltpu.CompilerParams(use_tc_tiling_on_sc=False),
    scratch_types=[
        pltpu.VMEM((G, GE), jnp.int32),
        pltpu.VMEM((G, GE), jnp.int32),
        pltpu.VMEM((NBUF, GE, DQ), jnp.float32),
        pltpu.VMEM_SHARED((NPAD, DQ), jnp.float32),
        pltpu.SemaphoreType.DMA((NBUF,)),
        pltpu.SemaphoreType.DMA,
    ],
)


def _deg_body(degidx, onerow, z16, dd, idx_v, ones_v, acc):
    c = lax.axis_index("c")
    s = lax.axis_index("s")
    pltpu.sync_copy(degidx.at[c, s], idx_v)
    pltpu.sync_copy(onerow, ones_v)
    pltpu.sync_copy(z16, acc.at[pl.ds(s * ZROWS, ZROWS)])
    plsc.subcore_barrier()

    def step(g, carry):
        pltpu.sync_copy(ones_v, acc.at[idx_v.at[g]], add=True)
        return carry

    lax.fori_loop(0, G, step, 0)
    plsc.subcore_barrier()
    pltpu.sync_copy(acc.at[pl.ds(s * ZROWS, ZROWS)],
                    dd.at[c, pl.ds(s * ZROWS, ZROWS)])


_deg = pl.kernel(
    _deg_body,
    out_type=jax.ShapeDtypeStruct((CORES, NPAD, 16), jnp.float32),
    mesh=_MESH,
    compiler_params=pltpu.CompilerParams(use_tc_tiling_on_sc=False),
    scratch_types=[
        pltpu.VMEM((G, GE), jnp.int32),
        pltpu.VMEM((GE, 16), jnp.float32),
        pltpu.VMEM_SHARED((NPAD, 16), jnp.float32),
    ],
)


def _nrm(d):
    return lax.rsqrt(jnp.maximum(d, 1.0))


def _matpair(aq, di, W, b):
    nd = _nrm(di[...])
    Wv = W[...]
    t = b[...]
    for q in range(4):
        t = t + jnp.dot(aq[q] * nd, Wv[DQ * q:DQ * (q + 1)],
                        preferred_element_type=jnp.float32)
    return t


def _quarters(o, gs):
    for q in range(4):
        o[q] = gs[:, DQ * q:DQ * (q + 1)]


def _mid_body(a0, a1, a2, a3, di, do, W, b, o):
    g = jnp.maximum(_matpair((a0[0], a1[0], a2[0], a3[0]), di, W, b), 0.0)
    _quarters(o, g * _nrm(do[...]))


def _ru3_body(a0, a1, a2, a3, di, do, W, b, h, z_o, hrs_o):
    ru = jax.nn.sigmoid(_matpair((a0[0], a1[0], a2[0], a3[0]), di, W, b))
    r = ru[:, :DH]
    z_o[...] = ru[:, DH:]
    hrs = h[...] * r * _nrm(do[...])
    hrs_o[0] = hrs[:, :DQ]
    hrs_o[1] = hrs[:, DQ:]


def _rh3_body(a0, a1, a2, a3, di, do, W, b, h, z, hn_o, hns_o):
    hc = jnp.tanh(_matpair((a0[0], a1[0], a2[0], a3[0]), di, W, b))
    zv = z[...]
    hn = zv * h[...] + (1.0 - zv) * hc
    hn_o[...] = hn
    hns = hn * _nrm(do[...])
    hns_o[0] = hns[:, :DQ]
    hns_o[1] = hns[:, DQ:]


def _pre_body(h0, Xs, do, h0s_o, Xss_o):
    ns = _nrm(do[...])
    h0s = h0[...] * ns
    h0s_o[0] = h0s[:, :DQ]
    h0s_o[1] = h0s[:, DQ:]
    xs = Xs[...] * ns[None]
    Xss_o[:, 0] = xs[..., :DQ]
    Xss_o[:, 1] = xs[..., DQ:]


_row = pl.BlockSpec((RBLK, DH), lambda i: (i, 0))
_col1 = pl.BlockSpec((RBLK, 1), lambda i: (i, 0))
_q = [pl.BlockSpec((1, RBLK, DQ), lambda i, q=q: (q, i, 0)) for q in range(4)]
_q4 = pl.BlockSpec((4, RBLK, DQ), lambda i: (0, i, 0))
_h2 = pl.BlockSpec((2, RBLK, DQ), lambda i: (0, i, 0))
_full = lambda shape: pl.BlockSpec(shape, lambda i: tuple(0 for _ in shape))
_GRID = N // RBLK

_mid = pl.pallas_call(
    _mid_body,
    grid=(_GRID,),
    in_specs=[*_q, _col1, _col1, _full((2 * DH, 2 * DH)), _full((1, 2 * DH))],
    out_specs=[_q4],
    out_shape=[jax.ShapeDtypeStruct((4, N, DQ), jnp.float32)],
)

_ru3 = pl.pallas_call(
    _ru3_body,
    grid=(_GRID,),
    in_specs=[*_q, _col1, _col1, _full((2 * DH, 2 * DH)),
              _full((1, 2 * DH)), _row],
    out_specs=[_row, _h2],
    out_shape=[jax.ShapeDtypeStruct((N, DH), jnp.float32),
               jax.ShapeDtypeStruct((2, N, DQ), jnp.float32)],
)

_rh3 = pl.pallas_call(
    _rh3_body,
    grid=(_GRID,),
    in_specs=[*_q, _col1, _col1, _full((2 * DH, DH)),
              _full((1, DH)), _row, _row],
    out_specs=[_row, _h2],
    out_shape=[jax.ShapeDtypeStruct((N, DH), jnp.float32),
               jax.ShapeDtypeStruct((2, N, DQ), jnp.float32)],
)

_pre = pl.pallas_call(
    _pre_body,
    grid=(_GRID,),
    in_specs=[_row, pl.BlockSpec((T, RBLK, DH), lambda i: (0, i, 0)), _col1],
    out_specs=[_h2, pl.BlockSpec((T, 2, RBLK, DQ), lambda i: (0, 0, i, 0))],
    out_shape=[jax.ShapeDtypeStruct((2, N, DQ), jnp.float32),
               jax.ShapeDtypeStruct((T, 2, N, DQ), jnp.float32)],
)


def kernel(Xs, h0, ru_W1, ru_b1, ru_W2, ru_b2, ru_W3, ru_b3,
           rh_W1, rh_b1, rh_W2, rh_b2, rh_W3, rh_b3, edge_index):
    src = edge_index[0].astype(jnp.int32)
    dst = edge_index[1].astype(jnp.int32)
    E = src.shape[0]
    pad = EPAD - E
    src_p0 = jnp.concatenate([src, jnp.zeros((pad,), jnp.int32)])
    src_pN = jnp.concatenate([src, jnp.full((pad,), N, jnp.int32)])
    dst_p = jnp.concatenate([dst, jnp.full((pad,), N, jnp.int32)])
    srcg = jnp.stack([src_p0 + q * N for q in range(4)])
    srcg = srcg.reshape(4, TILES, G, GE)
    dstg = dst_p.reshape(TILES, G, GE)
    degidx = jnp.stack([src_pN, dst_p]).reshape(CORES, TILES, G, GE)

    zrows = jnp.zeros((ZROWS, DQ), jnp.float32)
    z16 = jnp.zeros((ZROWS, 16), jnp.float32)
    onerow = jnp.zeros((GE, 16), jnp.float32).at[:, 0].set(1.0)

    dd = _deg(degidx, onerow, z16)
    deg_out = dd[0, :N, 0:1]
    deg_in = dd[1, :N, 0:1]

    b_ru1 = ru_b1.reshape(1, -1)
    b_ru2 = ru_b2.reshape(1, -1)
    b_ru3 = ru_b3.reshape(1, -1)
    b_rh1 = rh_b1.reshape(1, -1)
    b_rh2 = rh_b2.reshape(1, -1)
    b_rh3 = rh_b3.reshape(1, -1)

    h_s, Xss = _pre(h0, Xs, deg_out)

    def agg(table):
        a = _agg(table.reshape(-1, DQ), srcg, dstg, zrows)
        return (a, a, a, a)

    def gnn2(t0, t1, W1, b1, W2, b2):
        a = agg(jnp.concatenate([t0, t1], axis=0))
        (gg,) = _mid(*a, deg_in, deg_out, W1, b1)
        a = agg(gg)
        (gg,) = _mid(*a, deg_in, deg_out, W2, b2)
        return agg(gg)

    h = h0
    hs = []
    for t in range(T):
        xst = Xss[t]
        a = gnn2(h_s, xst, ru_W1, b_ru1, ru_W2, b_ru2)
        z, hrs = _ru3(*a, deg_in, deg_out, ru_W3, b_ru3, h)
        a = gnn2(hrs, xst, rh_W1, b_rh1, rh_W2, b_rh2)
        h, h_s = _rh3(*a, deg_in, deg_out, rh_W3, b_rh3, h, z)
        hs.append(h)
    return jnp.stack(hs)
